# Initial kernel scaffold; baseline (speedup 1.0000x reference)
import jax
import jax.numpy as jnp
from jax.experimental import pallas as pl


def kernel(X, keypoint_positions, kp_vis, noise_idx, bank, adj_mat):
    raise NotImplementedError("write your pallas kernel here")


# speedup vs baseline: 1.5651x; 1.5651x over previous
import numpy as np
import jax
import jax.numpy as jnp
from jax import lax
from jax.experimental import pallas as pl
from jax.experimental.pallas import tpu as pltpu

_T = 0.07
_N_ORIENT = 3
_NUM_NEG = 2048
_EPS_MASK = 100000.0
_MASK_NEG = float(-np.log(0.005))


def _loss_tc(featsT_ref, bank_ref, adjf_ref, visT_ref, num_ref, vis_ref, noise_ref):
    n = pl.program_id(0)

    @pl.when(n == 0)
    def _init():
        num_ref[...] = jnp.zeros_like(num_ref)
        vis_ref[...] = jnp.zeros_like(vis_ref)
        noise_ref[...] = jnp.zeros_like(noise_ref)

    kpT = featsT_ref[0, 0]
    nzT = featsT_ref[0, 1]

    def _norm(xT):
        s2 = jnp.sum(xT * xT, axis=0, keepdims=True)
        return xT / jnp.maximum(jnp.sqrt(s2), 1e-12)

    kpT = _norm(kpT)
    nzT = _norm(nzT)
    bank = bank_ref[...]

    sim = lax.dot_general(
        kpT, bank, (((0,), (1,)), ((), ())),
        preferred_element_type=jnp.float32,
        precision=lax.Precision.HIGHEST) / _T

    eye = (lax.broadcasted_iota(jnp.int32, (128, 128), 0)
           == lax.broadcasted_iota(jnp.int32, (128, 128), 1)).astype(jnp.float32)
    adjm = adjf_ref[...] * (1.0 - eye) * _EPS_MASK
    logits = jnp.concatenate(
        [sim[:, :128] - adjm,
         sim[:, 128:256] - adjm,
         sim[:, 256:384] - adjm,
         sim[:, 384:] - _MASK_NEG], axis=1)
    mx = jnp.max(logits, axis=1, keepdims=True)
    lse = jnp.log(jnp.sum(jnp.exp(logits - mx), axis=1, keepdims=True)) + mx
    lab = jnp.sum(logits[:, :128] * eye, axis=1, keepdims=True)
    nll = lse - lab
    visc = visT_ref[0]
    num_ref[...] += nll * visc
    vis_ref[...] += visc

    simn = lax.dot_general(
        nzT, bank[:384], (((0,), (1,)), ((), ())),
        preferred_element_type=jnp.float32,
        precision=lax.Precision.HIGHEST) / _T
    mxn = jnp.max(simn, axis=1, keepdims=True)
    lsen = jnp.log(jnp.sum(jnp.exp(simn - mxn), axis=1, keepdims=True)) + mxn
    noise_ref[...] += lsen


def _loss_from_featsT(featsT, bank, adjf, visT, interpret=False):
    num, vis, noise = pl.pallas_call(
        _loss_tc,
        grid=(16,),
        in_specs=[
            pl.BlockSpec((1, 2, 128, 128), lambda n: (n, 0, 0, 0)),
            pl.BlockSpec((2432, 128), lambda n: (0, 0)),
            pl.BlockSpec((128, 128), lambda n: (0, 0)),
            pl.BlockSpec((1, 128, 1), lambda n: (n, 0, 0)),
        ],
        out_specs=[pl.BlockSpec((128, 1), lambda n: (0, 0))] * 3,
        out_shape=[jax.ShapeDtypeStruct((128, 1), jnp.float32)] * 3,
        interpret=interpret,
    )(featsT, bank, adjf, visT)
    total = (jnp.sum(num) / jnp.clip(jnp.sum(vis), 1e-6)
             + jnp.sum(noise) / noise.size / 16.0)
    return total


def kernel(X, keypoint_positions, kp_vis, noise_idx, bank, adj_mat):
    N, C, H, W = X.shape
    K = keypoint_positions.shape[1]
    kp_idx = keypoint_positions[:, :, 0] * W + keypoint_positions[:, :, 1]
    all_idx = jnp.concatenate([kp_idx, noise_idx], axis=1)
    arr = jnp.take_along_axis(
        X.reshape(N, C, H * W), all_idx[:, None, :].astype(jnp.int32), axis=2)
    featsT = arr.reshape(N, C, 2, 128).transpose(0, 2, 1, 3)
    adjf = adj_mat[0].astype(jnp.float32)
    visT = kp_vis[:, :, None]
    return _loss_from_featsT(featsT, bank, adjf, visT)


# speedup vs baseline: 2.8233x; 1.8039x over previous
import functools

import numpy as np
import jax
import jax.numpy as jnp
from jax import lax
from jax.experimental import pallas as pl
from jax.experimental.pallas import tpu as pltpu
from jax.experimental.pallas import tpu_sc as plsc

_T = 0.07
_N_ORIENT = 3
_NUM_NEG = 2048
_EPS_MASK = 100000.0
_MASK_NEG = float(-np.log(0.005))


_HW = 128 * 128
_SAMP_PER_W = 128


def _sc_gather_impl(xflat, base_hbm, out_hbm, base_v, idx_v, feats_v, sem):
    wid = lax.axis_index("s") * 2 + lax.axis_index("c")
    start = wid * _SAMP_PER_W
    pltpu.sync_copy(base_hbm.at[pl.ds(start, _SAMP_PER_W)], base_v)

    for sb in range(_SAMP_PER_W // 16):
        seg = base_v[pl.ds(sb * 16, 16)]

        def build_body(c, carry, seg=seg, sb=sb):
            idx_v[c, pl.ds(sb * 16, 16)] = seg + c * _HW
            return carry

        lax.fori_loop(0, 128, build_body, 0)

    def gather_body(g, carry):
        copies = [
            pltpu.async_copy(
                xflat.at[idx_v.at[g * 16 + j]], feats_v.at[g * 16 + j], sem)
            for j in range(16)
        ]
        for cp in copies:
            cp.wait()
        return carry

    lax.fori_loop(0, 128 // 16, gather_body, 0)
    pltpu.sync_copy(feats_v, out_hbm.at[wid])


@functools.lru_cache(maxsize=1)
def _sc_gather_fn():
    return functools.partial(
        pl.kernel,
        mesh=plsc.VectorSubcoreMesh(core_axis_name="c", subcore_axis_name="s"),
        out_type=jax.ShapeDtypeStruct((32, 128, 128), jnp.float32),
        scratch_types=[
            pltpu.VMEM((_SAMP_PER_W,), jnp.int32),
            pltpu.VMEM((128, _SAMP_PER_W), jnp.int32),
            pltpu.VMEM((128, _SAMP_PER_W), jnp.float32),
            pltpu.SemaphoreType.DMA,
        ],
        compiler_params=pltpu.CompilerParams(use_tc_tiling_on_sc=False),
    )(_sc_gather_impl)


def _loss_tc(featsT_ref, bank_ref, adjf_ref, visT_ref, num_ref, vis_ref, noise_ref):
    n = pl.program_id(0)

    @pl.when(n == 0)
    def _init():
        num_ref[...] = jnp.zeros_like(num_ref)
        vis_ref[...] = jnp.zeros_like(vis_ref)
        noise_ref[...] = jnp.zeros_like(noise_ref)

    kpT = featsT_ref[0, 0]
    nzT = featsT_ref[0, 1]

    def _norm(xT):
        s2 = jnp.sum(xT * xT, axis=0, keepdims=True)
        return xT / jnp.maximum(jnp.sqrt(s2), 1e-12)

    kpT = _norm(kpT)
    nzT = _norm(nzT)
    bank = bank_ref[...]

    sim = lax.dot_general(
        kpT, bank, (((0,), (1,)), ((), ())),
        preferred_element_type=jnp.float32,
        precision=lax.Precision.HIGHEST) / _T

    eye = (lax.broadcasted_iota(jnp.int32, (128, 128), 0)
           == lax.broadcasted_iota(jnp.int32, (128, 128), 1)).astype(jnp.float32)
    adjm = adjf_ref[...] * (1.0 - eye) * _EPS_MASK
    logits = jnp.concatenate(
        [sim[:, :128] - adjm,
         sim[:, 128:256] - adjm,
         sim[:, 256:384] - adjm,
         sim[:, 384:] - _MASK_NEG], axis=1)
    mx = jnp.max(logits, axis=1, keepdims=True)
    lse = jnp.log(jnp.sum(jnp.exp(logits - mx), axis=1, keepdims=True)) + mx
    lab = jnp.sum(logits[:, :128] * eye, axis=1, keepdims=True)
    nll = lse - lab
    visc = visT_ref[0]
    num_ref[...] += nll * visc
    vis_ref[...] += visc

    simn = lax.dot_general(
        nzT, bank[:384], (((0,), (1,)), ((), ())),
        preferred_element_type=jnp.float32,
        precision=lax.Precision.HIGHEST) / _T
    mxn = jnp.max(simn, axis=1, keepdims=True)
    lsen = jnp.log(jnp.sum(jnp.exp(simn - mxn), axis=1, keepdims=True)) + mxn
    noise_ref[...] += lsen


def _loss_from_featsT(featsT, bank, adjf, visT, interpret=False):
    num, vis, noise = pl.pallas_call(
        _loss_tc,
        grid=(16,),
        in_specs=[
            pl.BlockSpec((1, 2, 128, 128), lambda n: (n, 0, 0, 0)),
            pl.BlockSpec((2432, 128), lambda n: (0, 0)),
            pl.BlockSpec((128, 128), lambda n: (0, 0)),
            pl.BlockSpec((1, 128, 1), lambda n: (n, 0, 0)),
        ],
        out_specs=[pl.BlockSpec((128, 1), lambda n: (0, 0))] * 3,
        out_shape=[jax.ShapeDtypeStruct((128, 1), jnp.float32)] * 3,
        interpret=interpret,
    )(featsT, bank, adjf, visT)
    total = (jnp.sum(num) / jnp.clip(jnp.sum(vis), 1e-6)
             + jnp.sum(noise) / noise.size / 16.0)
    return total


def kernel(X, keypoint_positions, kp_vis, noise_idx, bank, adj_mat):
    N, C, H, W = X.shape
    K = keypoint_positions.shape[1]
    kp_idx = keypoint_positions[:, :, 0] * W + keypoint_positions[:, :, 1]
    all_idx = jnp.concatenate([kp_idx, noise_idx], axis=1)
    base = (jnp.arange(N, dtype=jnp.int32)[:, None] * (C * H * W)
            + all_idx.astype(jnp.int32)).reshape(-1)
    feats4 = _sc_gather_fn()(X.reshape(N * C * H * W), base)
    featsT = feats4.reshape(N, 2, C, 128)
    adjf = adj_mat[0].astype(jnp.float32)
    visT = kp_vis[:, :, None]
    return _loss_from_featsT(featsT, bank, adjf, visT)


# speedup vs baseline: 3.3703x; 1.1938x over previous
import functools

import numpy as np
import jax
import jax.numpy as jnp
from jax import lax
from jax.experimental import pallas as pl
from jax.experimental.pallas import tpu as pltpu
from jax.experimental.pallas import tpu_sc as plsc

_T = 0.07
_N_ORIENT = 3
_NUM_NEG = 2048
_EPS_MASK = 100000.0
_MASK_NEG = float(-np.log(0.005))


_HW = 128 * 128
_SAMP_PER_W = 128


def _sc_gather_impl(xflat, base_hbm, out_hbm, base_v, idx_v, feats_v, sem):
    wid = lax.axis_index("s") * 2 + lax.axis_index("c")
    start = wid * _SAMP_PER_W
    pltpu.sync_copy(base_hbm.at[pl.ds(start, _SAMP_PER_W)], base_v)

    for sb in range(_SAMP_PER_W // 16):
        seg = base_v[pl.ds(sb * 16, 16)]

        def build_body(c, carry, seg=seg, sb=sb):
            idx_v[c, pl.ds(sb * 16, 16)] = seg + c * _HW
            return carry

        lax.fori_loop(0, 128, build_body, 0)

    def gather_body(g, carry):
        copies = [
            pltpu.async_copy(
                xflat.at[idx_v.at[g * 16 + j]], feats_v.at[g * 16 + j], sem)
            for j in range(16)
        ]
        for cp in copies:
            cp.wait()
        return carry

    lax.fori_loop(0, 128 // 16, gather_body, 0)
    pltpu.sync_copy(feats_v, out_hbm.at[wid])


@functools.lru_cache(maxsize=1)
def _sc_gather_fn():
    return functools.partial(
        pl.kernel,
        mesh=plsc.VectorSubcoreMesh(core_axis_name="c", subcore_axis_name="s"),
        out_type=jax.ShapeDtypeStruct((32, 128, 128), jnp.float32),
        scratch_types=[
            pltpu.VMEM((_SAMP_PER_W,), jnp.int32),
            pltpu.VMEM((128, _SAMP_PER_W), jnp.int32),
            pltpu.VMEM((128, _SAMP_PER_W), jnp.float32),
            pltpu.SemaphoreType.DMA,
        ],
        compiler_params=pltpu.CompilerParams(use_tc_tiling_on_sc=False),
    )(_sc_gather_impl)


def _loss_tc(featsT_ref, bank_ref, adjf_ref, visT_ref, num_ref, vis_ref, noise_ref):
    n = pl.program_id(0)

    @pl.when(n == 0)
    def _init():
        num_ref[...] = jnp.zeros_like(num_ref)
        vis_ref[...] = jnp.zeros_like(vis_ref)
        noise_ref[...] = jnp.zeros_like(noise_ref)

    kpT = featsT_ref[0, 0]
    nzT = featsT_ref[0, 1]

    def _norm(xT):
        s2 = jnp.sum(xT * xT, axis=0, keepdims=True)
        return xT / jnp.maximum(jnp.sqrt(s2), 1e-12)

    kpT = _norm(kpT)
    nzT = _norm(nzT)
    bank = bank_ref[...]

    sim = lax.dot_general(
        kpT, bank, (((0,), (1,)), ((), ())),
        preferred_element_type=jnp.float32,
        precision=lax.Precision.DEFAULT) / _T

    eye = (lax.broadcasted_iota(jnp.int32, (128, 128), 0)
           == lax.broadcasted_iota(jnp.int32, (128, 128), 1)).astype(jnp.float32)
    adjm = adjf_ref[...] * (1.0 - eye) * _EPS_MASK
    logits = jnp.concatenate(
        [sim[:, :128] - adjm,
         sim[:, 128:256] - adjm,
         sim[:, 256:384] - adjm,
         sim[:, 384:] - _MASK_NEG], axis=1)
    mx = jnp.max(logits, axis=1, keepdims=True)
    lse = jnp.log(jnp.sum(jnp.exp(logits - mx), axis=1, keepdims=True)) + mx
    lab = jnp.sum(logits[:, :128] * eye, axis=1, keepdims=True)
    nll = lse - lab
    visc = visT_ref[0]
    num_ref[...] += nll * visc
    vis_ref[...] += visc

    simn = lax.dot_general(
        nzT, bank[:384], (((0,), (1,)), ((), ())),
        preferred_element_type=jnp.float32,
        precision=lax.Precision.DEFAULT) / _T
    mxn = jnp.max(simn, axis=1, keepdims=True)
    lsen = jnp.log(jnp.sum(jnp.exp(simn - mxn), axis=1, keepdims=True)) + mxn
    noise_ref[...] += lsen


def _loss_from_featsT(featsT, bank, adjf, visT, interpret=False):
    num, vis, noise = pl.pallas_call(
        _loss_tc,
        grid=(16,),
        in_specs=[
            pl.BlockSpec((1, 2, 128, 128), lambda n: (n, 0, 0, 0)),
            pl.BlockSpec((2432, 128), lambda n: (0, 0)),
            pl.BlockSpec((128, 128), lambda n: (0, 0)),
            pl.BlockSpec((1, 128, 1), lambda n: (n, 0, 0)),
        ],
        out_specs=[pl.BlockSpec((128, 1), lambda n: (0, 0))] * 3,
        out_shape=[jax.ShapeDtypeStruct((128, 1), jnp.float32)] * 3,
        interpret=interpret,
    )(featsT, bank, adjf, visT)
    total = (jnp.sum(num) / jnp.clip(jnp.sum(vis), 1e-6)
             + jnp.sum(noise) / noise.size / 16.0)
    return total


def kernel(X, keypoint_positions, kp_vis, noise_idx, bank, adj_mat):
    N, C, H, W = X.shape
    K = keypoint_positions.shape[1]
    kp_idx = keypoint_positions[:, :, 0] * W + keypoint_positions[:, :, 1]
    all_idx = jnp.concatenate([kp_idx, noise_idx], axis=1)
    base = (jnp.arange(N, dtype=jnp.int32)[:, None] * (C * H * W)
            + all_idx.astype(jnp.int32)).reshape(-1)
    feats4 = _sc_gather_fn()(X.reshape(N * C * H * W), base)
    featsT = feats4.reshape(N, 2, C, 128)
    adjf = adj_mat[0].astype(jnp.float32)
    visT = kp_vis[:, :, None]
    return _loss_from_featsT(featsT, bank, adjf, visT)


# speedup vs baseline: 3.7567x; 1.1147x over previous
import functools

import numpy as np
import jax
import jax.numpy as jnp
from jax import lax
from jax.experimental import pallas as pl
from jax.experimental.pallas import tpu as pltpu
from jax.experimental.pallas import tpu_sc as plsc

_T = 0.07
_N_ORIENT = 3
_NUM_NEG = 2048
_EPS_MASK = 100000.0
_MASK_NEG = float(-np.log(0.005))


_HW = 128 * 128
_SAMP_PER_W = 128


def _sc_gather_impl(xflat, base_hbm, out_hbm, base_v, idx_v, feats_v, sem):
    wid = lax.axis_index("s") * 2 + lax.axis_index("c")
    start = wid * _SAMP_PER_W
    pltpu.sync_copy(base_hbm.at[pl.ds(start, _SAMP_PER_W)], base_v)

    def build_body(c, carry):
        coff = c * _HW
        for sb in range(_SAMP_PER_W // 16):
            idx_v[c, pl.ds(sb * 16, 16)] = base_v[pl.ds(sb * 16, 16)] + coff
        return carry

    lax.fori_loop(0, 128, build_body, 0)

    def gather_body(g, carry):
        @pl.when(g > 0)
        def _drain_prev():
            for j in range(16):
                pltpu.make_async_copy(
                    xflat.at[idx_v.at[(g - 1) * 16 + j]],
                    feats_v.at[(g - 1) * 16 + j], sem).wait()

        for j in range(16):
            pltpu.async_copy(
                xflat.at[idx_v.at[g * 16 + j]], feats_v.at[g * 16 + j], sem)
        return carry

    lax.fori_loop(0, 128 // 16, gather_body, 0)
    for j in range(16):
        pltpu.make_async_copy(
            xflat.at[idx_v.at[112 + j]], feats_v.at[112 + j], sem).wait()
    pltpu.sync_copy(feats_v, out_hbm.at[wid])


@functools.lru_cache(maxsize=1)
def _sc_gather_fn():
    return functools.partial(
        pl.kernel,
        mesh=plsc.VectorSubcoreMesh(core_axis_name="c", subcore_axis_name="s"),
        out_type=jax.ShapeDtypeStruct((32, 128, 128), jnp.float32),
        scratch_types=[
            pltpu.VMEM((_SAMP_PER_W,), jnp.int32),
            pltpu.VMEM((128, _SAMP_PER_W), jnp.int32),
            pltpu.VMEM((128, _SAMP_PER_W), jnp.float32),
            pltpu.SemaphoreType.DMA,
        ],
        compiler_params=pltpu.CompilerParams(use_tc_tiling_on_sc=False),
    )(_sc_gather_impl)


def _loss_tc(featsT_ref, bank_ref, adjf_ref, visT_ref, out_ref,
             num_ref, vis_ref, noise_ref):
    n = pl.program_id(0)

    @pl.when(n == 0)
    def _init():
        num_ref[...] = jnp.zeros_like(num_ref)
        vis_ref[...] = jnp.zeros_like(vis_ref)
        noise_ref[...] = jnp.zeros_like(noise_ref)

    kpT = featsT_ref[0, 0]
    nzT = featsT_ref[0, 1]

    def _norm(xT):
        s2 = jnp.sum(xT * xT, axis=0, keepdims=True)
        return xT / jnp.maximum(jnp.sqrt(s2), 1e-12)

    kpT = _norm(kpT)
    nzT = _norm(nzT)
    bank = bank_ref[...]

    sim = lax.dot_general(
        kpT, bank, (((0,), (1,)), ((), ())),
        preferred_element_type=jnp.float32,
        precision=lax.Precision.DEFAULT) / _T

    eye = (lax.broadcasted_iota(jnp.int32, (128, 128), 0)
           == lax.broadcasted_iota(jnp.int32, (128, 128), 1)).astype(jnp.float32)
    adjm = adjf_ref[...] * (1.0 - eye) * _EPS_MASK
    logits = jnp.concatenate(
        [sim[:, :128] - adjm,
         sim[:, 128:256] - adjm,
         sim[:, 256:384] - adjm,
         sim[:, 384:] - _MASK_NEG], axis=1)
    mx = jnp.max(logits, axis=1, keepdims=True)
    lse = jnp.log(jnp.sum(jnp.exp(logits - mx), axis=1, keepdims=True)) + mx
    lab = jnp.sum(logits[:, :128] * eye, axis=1, keepdims=True)
    nll = lse - lab
    visc = visT_ref[0]
    num_ref[...] += nll * visc
    vis_ref[...] += visc

    simn = lax.dot_general(
        nzT, bank[:384], (((0,), (1,)), ((), ())),
        preferred_element_type=jnp.float32,
        precision=lax.Precision.DEFAULT) / _T
    mxn = jnp.max(simn, axis=1, keepdims=True)
    lsen = jnp.log(jnp.sum(jnp.exp(simn - mxn), axis=1, keepdims=True)) + mxn
    noise_ref[...] += lsen

    @pl.when(n == 15)
    def _finish():
        total = (jnp.sum(num_ref[...]) / jnp.clip(jnp.sum(vis_ref[...]), 1e-6)
                 + jnp.sum(noise_ref[...]) / 2048.0)
        out_ref[...] = jnp.full((1, 1), total, jnp.float32)


def _loss_from_featsT(featsT, bank, adjf, visT, interpret=False):
    out = pl.pallas_call(
        _loss_tc,
        grid=(16,),
        in_specs=[
            pl.BlockSpec((1, 2, 128, 128), lambda n: (n, 0, 0, 0)),
            pl.BlockSpec((2432, 128), lambda n: (0, 0)),
            pl.BlockSpec((128, 128), lambda n: (0, 0)),
            pl.BlockSpec((1, 128, 1), lambda n: (n, 0, 0)),
        ],
        out_specs=pl.BlockSpec((1, 1), lambda n: (0, 0)),
        out_shape=jax.ShapeDtypeStruct((1, 1), jnp.float32),
        scratch_shapes=[pltpu.VMEM((128, 1), jnp.float32)] * 3,
        interpret=interpret,
    )(featsT, bank, adjf, visT)
    return out[0, 0]


def kernel(X, keypoint_positions, kp_vis, noise_idx, bank, adj_mat):
    N, C, H, W = X.shape
    K = keypoint_positions.shape[1]
    kp_idx = keypoint_positions[:, :, 0] * W + keypoint_positions[:, :, 1]
    all_idx = jnp.concatenate([kp_idx, noise_idx], axis=1)
    base = (jnp.arange(N, dtype=jnp.int32)[:, None] * (C * H * W)
            + all_idx.astype(jnp.int32)).reshape(-1)
    feats4 = _sc_gather_fn()(X.reshape(N * C * H * W), base)
    featsT = feats4.reshape(N, 2, C, 128)
    adjf = adj_mat[0].astype(jnp.float32)
    visT = kp_vis[:, :, None]
    return _loss_from_featsT(featsT, bank, adjf, visT)


# speedup vs baseline: 4.0667x; 1.0825x over previous
import functools

import numpy as np
import jax
import jax.numpy as jnp
from jax import lax
from jax.experimental import pallas as pl
from jax.experimental.pallas import tpu as pltpu
from jax.experimental.pallas import tpu_sc as plsc

_T = 0.07
_N_ORIENT = 3
_NUM_NEG = 2048
_EPS_MASK = 100000.0
_MASK_NEG = float(-np.log(0.005))


_HW = 128 * 128
_SAMP_PER_W = 128


def _sc_gather_impl(xflat, base_hbm, out_hbm, base_v, idx_v, feats_v, sem):
    wid = lax.axis_index("s") * 2 + lax.axis_index("c")
    start = wid * _SAMP_PER_W
    pltpu.sync_copy(base_hbm.at[pl.ds(start, _SAMP_PER_W)], base_v)

    def build_body(c, carry):
        coff = c * _HW
        for sb in range(_SAMP_PER_W // 16):
            idx_v[pl.ds(c * _SAMP_PER_W + sb * 16, 16)] = (
                base_v[pl.ds(sb * 16, 16)] + coff)
        return carry

    lax.fori_loop(0, 128, build_body, 0)

    pltpu.async_copy(xflat.at[idx_v], feats_v, sem).wait()
    pltpu.sync_copy(feats_v, out_hbm.at[wid])


@functools.lru_cache(maxsize=1)
def _sc_gather_fn():
    return functools.partial(
        pl.kernel,
        mesh=plsc.VectorSubcoreMesh(core_axis_name="c", subcore_axis_name="s"),
        out_type=jax.ShapeDtypeStruct((32, 128 * 128), jnp.float32),
        scratch_types=[
            pltpu.VMEM((_SAMP_PER_W,), jnp.int32),
            pltpu.VMEM((128 * _SAMP_PER_W,), jnp.int32),
            pltpu.VMEM((128 * _SAMP_PER_W,), jnp.float32),
            pltpu.SemaphoreType.DMA,
        ],
        compiler_params=pltpu.CompilerParams(use_tc_tiling_on_sc=False),
    )(_sc_gather_impl)


def _loss_tc(featsT_ref, bank_ref, adjf_ref, visT_ref, out_ref,
             num_ref, vis_ref, noise_ref):
    n = pl.program_id(0)

    @pl.when(n == 0)
    def _init():
        num_ref[...] = jnp.zeros_like(num_ref)
        vis_ref[...] = jnp.zeros_like(vis_ref)
        noise_ref[...] = jnp.zeros_like(noise_ref)

    kpT = featsT_ref[0, 0]
    nzT = featsT_ref[0, 1]

    def _norm(xT):
        s2 = jnp.sum(xT * xT, axis=0, keepdims=True)
        return xT / jnp.maximum(jnp.sqrt(s2), 1e-12)

    kpT = _norm(kpT)
    nzT = _norm(nzT)
    bank = bank_ref[...]

    sim = lax.dot_general(
        kpT, bank, (((0,), (1,)), ((), ())),
        preferred_element_type=jnp.float32,
        precision=lax.Precision.DEFAULT) / _T

    eye = (lax.broadcasted_iota(jnp.int32, (128, 128), 0)
           == lax.broadcasted_iota(jnp.int32, (128, 128), 1)).astype(jnp.float32)
    adjm = adjf_ref[...] * (1.0 - eye) * _EPS_MASK
    logits = jnp.concatenate(
        [sim[:, :128] - adjm,
         sim[:, 128:256] - adjm,
         sim[:, 256:384] - adjm,
         sim[:, 384:] - _MASK_NEG], axis=1)
    mx = jnp.max(logits, axis=1, keepdims=True)
    lse = jnp.log(jnp.sum(jnp.exp(logits - mx), axis=1, keepdims=True)) + mx
    lab = jnp.sum(logits[:, :128] * eye, axis=1, keepdims=True)
    nll = lse - lab
    visc = visT_ref[0]
    num_ref[...] += nll * visc
    vis_ref[...] += visc

    simn = lax.dot_general(
        nzT, bank[:384], (((0,), (1,)), ((), ())),
        preferred_element_type=jnp.float32,
        precision=lax.Precision.DEFAULT) / _T
    mxn = jnp.max(simn, axis=1, keepdims=True)
    lsen = jnp.log(jnp.sum(jnp.exp(simn - mxn), axis=1, keepdims=True)) + mxn
    noise_ref[...] += lsen

    @pl.when(n == 15)
    def _finish():
        total = (jnp.sum(num_ref[...]) / jnp.clip(jnp.sum(vis_ref[...]), 1e-6)
                 + jnp.sum(noise_ref[...]) / 2048.0)
        out_ref[...] = jnp.full((1, 1), total, jnp.float32)


def _loss_from_featsT(featsT, bank, adjf, visT, interpret=False):
    out = pl.pallas_call(
        _loss_tc,
        grid=(16,),
        in_specs=[
            pl.BlockSpec((1, 2, 128, 128), lambda n: (n, 0, 0, 0)),
            pl.BlockSpec((2432, 128), lambda n: (0, 0)),
            pl.BlockSpec((128, 128), lambda n: (0, 0)),
            pl.BlockSpec((1, 128, 1), lambda n: (n, 0, 0)),
        ],
        out_specs=pl.BlockSpec((1, 1), lambda n: (0, 0)),
        out_shape=jax.ShapeDtypeStruct((1, 1), jnp.float32),
        scratch_shapes=[pltpu.VMEM((128, 1), jnp.float32)] * 3,
        interpret=interpret,
    )(featsT, bank, adjf, visT)
    return out[0, 0]


def kernel(X, keypoint_positions, kp_vis, noise_idx, bank, adj_mat):
    N, C, H, W = X.shape
    K = keypoint_positions.shape[1]
    kp_idx = keypoint_positions[:, :, 0] * W + keypoint_positions[:, :, 1]
    all_idx = jnp.concatenate([kp_idx, noise_idx], axis=1)
    base = (jnp.arange(N, dtype=jnp.int32)[:, None] * (C * H * W)
            + all_idx.astype(jnp.int32)).reshape(-1)
    feats4 = _sc_gather_fn()(X.reshape(N * C * H * W), base)
    featsT = feats4.reshape(N, 2, C, 128)
    adjf = adj_mat[0].astype(jnp.float32)
    visT = kp_vis[:, :, None]
    return _loss_from_featsT(featsT, bank, adjf, visT)


# speedup vs baseline: 4.0986x; 1.0078x over previous
import functools

import numpy as np
import jax
import jax.numpy as jnp
from jax import lax
from jax.experimental import pallas as pl
from jax.experimental.pallas import tpu as pltpu
from jax.experimental.pallas import tpu_sc as plsc

_T = 0.07
_N_ORIENT = 3
_NUM_NEG = 2048
_EPS_MASK = 100000.0
_MASK_NEG = float(-np.log(0.005))


_HW = 128 * 128
_SAMP_PER_W = 128


_CHW = 128 * _HW


def _sc_gather_impl(xflat, kp_hbm, noise_hbm, out_hbm,
                    base_v, kpbuf_v, idx_v, feats_v, sem):
    wid = lax.axis_index("s") * 2 + lax.axis_index("c")
    n = wid // 2
    kpn = wid % 2
    noff = n * _CHW

    @pl.when(kpn == 0)
    def _kp_base():
        pltpu.sync_copy(kp_hbm.at[pl.ds(n * 256, 256)], kpbuf_v)
        for sb in range(8):
            ii = (lax.broadcasted_iota(jnp.int32, (16,), 0) + sb * 16) * 2
            y = plsc.load_gather(kpbuf_v, [ii])
            x = plsc.load_gather(kpbuf_v, [ii + 1])
            base_v[pl.ds(sb * 16, 16)] = y * 128 + x + noff

    @pl.when(kpn == 1)
    def _noise_base():
        pltpu.sync_copy(noise_hbm.at[pl.ds(n * 128, 128)], base_v)
        for sb in range(8):
            base_v[pl.ds(sb * 16, 16)] = base_v[pl.ds(sb * 16, 16)] + noff

    def build_body(c, carry):
        coff = c * _HW
        for sb in range(_SAMP_PER_W // 16):
            idx_v[pl.ds(c * _SAMP_PER_W + sb * 16, 16)] = (
                base_v[pl.ds(sb * 16, 16)] + coff)
        return carry

    lax.fori_loop(0, 128, build_body, 0)

    pltpu.async_copy(xflat.at[idx_v], feats_v, sem).wait()
    pltpu.sync_copy(feats_v, out_hbm.at[wid])


@functools.lru_cache(maxsize=1)
def _sc_gather_fn():
    return functools.partial(
        pl.kernel,
        mesh=plsc.VectorSubcoreMesh(core_axis_name="c", subcore_axis_name="s"),
        out_type=jax.ShapeDtypeStruct((32, 128 * 128), jnp.float32),
        scratch_types=[
            pltpu.VMEM((_SAMP_PER_W,), jnp.int32),
            pltpu.VMEM((2 * _SAMP_PER_W,), jnp.int32),
            pltpu.VMEM((128 * _SAMP_PER_W,), jnp.int32),
            pltpu.VMEM((128 * _SAMP_PER_W,), jnp.float32),
            pltpu.SemaphoreType.DMA,
        ],
        compiler_params=pltpu.CompilerParams(
            use_tc_tiling_on_sc=False, needs_layout_passes=False),
    )(_sc_gather_impl)


def _loss_tc(featsT_ref, bank_ref, adjf_ref, visT_ref, out_ref,
             num_ref, vis_ref, noise_ref):
    n = pl.program_id(0)

    @pl.when(n == 0)
    def _init():
        num_ref[...] = jnp.zeros_like(num_ref)
        vis_ref[...] = jnp.zeros_like(vis_ref)
        noise_ref[...] = jnp.zeros_like(noise_ref)

    kpT = featsT_ref[0, 0]
    nzT = featsT_ref[0, 1]

    def _norm(xT):
        s2 = jnp.sum(xT * xT, axis=0, keepdims=True)
        return xT / jnp.maximum(jnp.sqrt(s2), 1e-12)

    kpT = _norm(kpT)
    nzT = _norm(nzT)
    bank = bank_ref[...]

    sim = lax.dot_general(
        kpT, bank, (((0,), (1,)), ((), ())),
        preferred_element_type=jnp.float32,
        precision=lax.Precision.DEFAULT) / _T

    eye = (lax.broadcasted_iota(jnp.int32, (128, 128), 0)
           == lax.broadcasted_iota(jnp.int32, (128, 128), 1)).astype(jnp.float32)
    adjm = adjf_ref[...] * (1.0 - eye) * _EPS_MASK
    l3 = jnp.concatenate([sim[:, :128] - adjm,
                          sim[:, 128:256] - adjm,
                          sim[:, 256:384] - adjm], axis=1)
    neg = sim[:, 384:]
    mx = jnp.maximum(jnp.max(l3, axis=1, keepdims=True),
                     jnp.max(neg, axis=1, keepdims=True) - _MASK_NEG)
    se = (jnp.sum(jnp.exp(l3 - mx), axis=1, keepdims=True)
          + jnp.sum(jnp.exp(neg - (mx + _MASK_NEG)), axis=1, keepdims=True))
    lse = jnp.log(se) + mx
    lab = jnp.sum(sim[:, :128] * eye, axis=1, keepdims=True)
    nll = lse - lab
    visc = visT_ref[0]
    num_ref[...] += nll * visc
    vis_ref[...] += visc

    simn = lax.dot_general(
        nzT, bank[:384], (((0,), (1,)), ((), ())),
        preferred_element_type=jnp.float32,
        precision=lax.Precision.DEFAULT) / _T
    mxn = jnp.max(simn, axis=1, keepdims=True)
    lsen = jnp.log(jnp.sum(jnp.exp(simn - mxn), axis=1, keepdims=True)) + mxn
    noise_ref[...] += lsen

    @pl.when(n == 15)
    def _finish():
        total = (jnp.sum(num_ref[...]) / jnp.clip(jnp.sum(vis_ref[...]), 1e-6)
                 + jnp.sum(noise_ref[...]) / 2048.0)
        out_ref[...] = jnp.full((1, 1), total, jnp.float32)


def _loss_from_featsT(featsT, bank, adjf, visT, interpret=False):
    out = pl.pallas_call(
        _loss_tc,
        grid=(16,),
        in_specs=[
            pl.BlockSpec((1, 2, 128, 128), lambda n: (n, 0, 0, 0)),
            pl.BlockSpec((2432, 128), lambda n: (0, 0)),
            pl.BlockSpec((128, 128), lambda n: (0, 0)),
            pl.BlockSpec((1, 128, 1), lambda n: (n, 0, 0)),
        ],
        out_specs=pl.BlockSpec((1, 1), lambda n: (0, 0)),
        out_shape=jax.ShapeDtypeStruct((1, 1), jnp.float32),
        scratch_shapes=[pltpu.VMEM((128, 1), jnp.float32)] * 3,
        interpret=interpret,
    )(featsT, bank, adjf, visT)
    return out[0, 0]


def kernel(X, keypoint_positions, kp_vis, noise_idx, bank, adj_mat):
    N, C, H, W = X.shape
    kp_flat = keypoint_positions.reshape(-1).astype(jnp.int32)
    nz_flat = noise_idx.reshape(-1).astype(jnp.int32)
    feats4 = _sc_gather_fn()(X.reshape(N * C * H * W), kp_flat, nz_flat)
    featsT = feats4.reshape(N, 2, C, 128)
    adjf = adj_mat[0].astype(jnp.float32)
    visT = kp_vis[:, :, None]
    return _loss_from_featsT(featsT, bank, adjf, visT)
